# Initial kernel scaffold; baseline (speedup 1.0000x reference)
import jax
import jax.numpy as jnp
from jax.experimental import pallas as pl


def kernel(x, edge_index, edge_w, temp, W0, b0, W1, b1, Ew0, eb0, Ew1, eb1, Ew2, eb2):
    raise NotImplementedError("write your pallas kernel here")


# speedup vs baseline: 3.7512x; 3.7512x over previous
import functools

import jax
import jax.numpy as jnp
from jax import lax
from jax.experimental import pallas as pl
from jax.experimental.pallas import tpu as pltpu
from jax.experimental.pallas import tpu_sc as plsc

N = 10000
D = 128
E = 320000

NC = 2
NS = 16
DH = D // NC
EPS = E // NS
CHUNK = 80
NCHUNK = EPS // CHUNK
HALF = NCHUNK // 2
N_PAD = 10240
RPT = N_PAD // NS
ZR = 128

BN = 2000
GRID = N // BN


def _pre_body(x_ref, W_ref, b_ref, Ew_ref, eb_ref, h_ref, e_ref):
    xb = x_ref[...]
    h = jnp.dot(xb, W_ref[...], preferred_element_type=jnp.float32) + b_ref[...]
    h_ref[0] = h[:, :DH]
    h_ref[1] = h[:, DH:]
    e_ref[...] = (
        jnp.dot(xb, Ew_ref[...], preferred_element_type=jnp.float32) + eb_ref[...]
    )


_tc_pre = pl.pallas_call(
    _pre_body,
    grid=(GRID,),
    in_specs=[
        pl.BlockSpec((BN, D), lambda i: (i, 0)),
        pl.BlockSpec((D, D), lambda i: (0, 0)),
        pl.BlockSpec((1, D), lambda i: (0, 0)),
        pl.BlockSpec((D, 1), lambda i: (0, 0)),
        pl.BlockSpec((1, 1), lambda i: (0, 0)),
    ],
    out_specs=[
        pl.BlockSpec((2, BN, DH), lambda i: (0, i, 0)),
        pl.BlockSpec((BN, 1), lambda i: (i, 0)),
    ],
    out_shape=[
        jax.ShapeDtypeStruct((2, N, DH), jnp.float32),
        jax.ShapeDtypeStruct((N, 1), jnp.float32),
    ],
)


def _mid_body(p_ref, W_ref, b_ref, Ew_ref, eb_ref, e0_ref, h_ref, e_ref):
    s = jnp.concatenate([p_ref[0], p_ref[1]], axis=-1)
    x1 = jnp.where(s >= 0, s, 0.01 * s)
    h = jnp.dot(x1, W_ref[...], preferred_element_type=jnp.float32) + b_ref[...]
    h_ref[0] = h[:, :DH]
    h_ref[1] = h[:, DH:]
    e_ref[...] = (
        e0_ref[...]
        + jnp.dot(x1, Ew_ref[...], preferred_element_type=jnp.float32)
        + eb_ref[...]
    )


_tc_mid = pl.pallas_call(
    _mid_body,
    grid=(GRID,),
    in_specs=[
        pl.BlockSpec((2, BN, DH), lambda i: (0, i, 0)),
        pl.BlockSpec((D, D), lambda i: (0, 0)),
        pl.BlockSpec((1, D), lambda i: (0, 0)),
        pl.BlockSpec((D, 1), lambda i: (0, 0)),
        pl.BlockSpec((1, 1), lambda i: (0, 0)),
        pl.BlockSpec((BN, 1), lambda i: (i, 0)),
    ],
    out_specs=[
        pl.BlockSpec((2, BN, DH), lambda i: (0, i, 0)),
        pl.BlockSpec((BN, 1), lambda i: (i, 0)),
    ],
    out_shape=[
        jax.ShapeDtypeStruct((2, N, DH), jnp.float32),
        jax.ShapeDtypeStruct((N, 1), jnp.float32),
    ],
)


def _post_body(p_ref, Ew_ref, eb_ref, e1_ref, x_ref, e_ref):
    s = jnp.concatenate([p_ref[0], p_ref[1]], axis=-1)
    x2 = jnp.where(s >= 0, s, 0.01 * s)
    x_ref[...] = x2
    e_ref[...] = (
        e1_ref[...]
        + jnp.dot(x2, Ew_ref[...], preferred_element_type=jnp.float32)
        + eb_ref[...]
    )


_tc_post = pl.pallas_call(
    _post_body,
    grid=(GRID,),
    in_specs=[
        pl.BlockSpec((2, BN, DH), lambda i: (0, i, 0)),
        pl.BlockSpec((D, 1), lambda i: (0, 0)),
        pl.BlockSpec((1, 1), lambda i: (0, 0)),
        pl.BlockSpec((BN, 1), lambda i: (i, 0)),
    ],
    out_specs=[
        pl.BlockSpec((BN, D), lambda i: (i, 0)),
        pl.BlockSpec((BN, 1), lambda i: (i, 0)),
    ],
    out_shape=[
        jax.ShapeDtypeStruct((N, D), jnp.float32),
        jax.ShapeDtypeStruct((N, 1), jnp.float32),
    ],
)


_sc_mesh = plsc.VectorSubcoreMesh(core_axis_name="c", subcore_axis_name="s")


@functools.partial(
    pl.kernel,
    mesh=_sc_mesh,
    compiler_params=pltpu.CompilerParams(use_tc_tiling_on_sc=False),
    out_type=jax.ShapeDtypeStruct((NC, N_PAD, DH), jnp.float32),
    scratch_types=[
        pltpu.VMEM((NCHUNK, CHUNK), jnp.int32),
        pltpu.VMEM((NCHUNK, CHUNK), jnp.int32),
        pltpu.VMEM((NCHUNK, CHUNK), jnp.float32),
        pltpu.VMEM((2, CHUNK, DH), jnp.float32),
        pltpu.VMEM((ZR, DH), jnp.float32),
        pltpu.VMEM_SHARED((N_PAD, DH), jnp.float32),
        pltpu.SemaphoreType.DMA,
    ],
)
def _edge_pass(h_hbm, src_hbm, dst_hbm, w_hbm, out_hbm,
               srcv, dstv, wv, rows, zbuf, acc, gsem):
    c = lax.axis_index("c")
    s = lax.axis_index("s")

    def zfill(i, carry):
        for l in range(DH // 16):
            zbuf[i, pl.ds(l * 16, 16)] = jnp.zeros((16,), jnp.float32)
        return carry

    lax.fori_loop(0, ZR, zfill, 0)
    base = s * RPT
    for k in range(RPT // ZR):
        pltpu.sync_copy(zbuf, acc.at[pl.ds(base + k * ZR, ZR)])
    plsc.subcore_barrier()

    pltpu.sync_copy(src_hbm.at[s], srcv)
    pltpu.sync_copy(dst_hbm.at[s], dstv)
    pltpu.sync_copy(w_hbm.at[s], wv)

    hc = h_hbm.at[c]

    def gather_start(j, buf):
        return pltpu.async_copy(hc.at[srcv.at[j]], rows.at[buf], gsem)

    def gather_wait(j, buf):
        pltpu.make_async_copy(hc.at[srcv.at[j]], rows.at[buf], gsem).wait()

    def process(j, buf):
        def group(g, carry):
            w16 = wv[j, pl.ds(g * 16, 16)]
            for jj in range(16):
                e = g * 16 + jj
                w_e = jnp.full((16,), w16[jj], jnp.float32)
                for l in range(DH // 16):
                    sl = pl.ds(l * 16, 16)
                    rows[buf, e, sl] = rows[buf, e, sl] * w_e
            return carry

        lax.fori_loop(0, CHUNK // 16, group, 0)
        pltpu.sync_copy(rows.at[buf], acc.at[dstv.at[j]], add=True)

    gather_start(0, 0)

    def body(i, carry):
        j0 = 2 * i
        gather_start(j0 + 1, 1)
        gather_wait(j0, 0)
        process(j0, 0)
        gather_start(j0 + 2, 0)
        gather_wait(j0 + 1, 1)
        process(j0 + 1, 1)
        return carry

    lax.fori_loop(0, HALF - 1, body, 0)
    gather_start(NCHUNK - 1, 1)
    gather_wait(NCHUNK - 2, 0)
    process(NCHUNK - 2, 0)
    gather_wait(NCHUNK - 1, 1)
    process(NCHUNK - 1, 1)

    plsc.subcore_barrier()
    pltpu.sync_copy(acc.at[pl.ds(base, RPT)], out_hbm.at[c, pl.ds(base, RPT)])


def kernel(x, edge_index, edge_w, temp, W0, b0, W1, b1, Ew0, eb0, Ew1, eb1, Ew2, eb2):
    src3 = edge_index[0].reshape(NS, NCHUNK, CHUNK)
    dst3 = edge_index[1].reshape(NS, NCHUNK, CHUNK)
    w3 = edge_w.reshape(NS, NCHUNK, CHUNK)

    b0r = b0.reshape(1, D)
    b1r = b1.reshape(1, D)
    Ew0s = Ew0 * temp[0]
    eb0s = (eb0 * temp[0]).reshape(1, 1)
    Ew1s = Ew1 * temp[1]
    eb1s = (eb1 * temp[1]).reshape(1, 1)
    Ew2s = Ew2 * temp[2]
    eb2s = (eb2 * temp[2]).reshape(1, 1)

    h0, e0 = _tc_pre(x, W0, b0r, Ew0s, eb0s)
    p0 = _edge_pass(h0, src3, dst3, w3)
    h1, e1 = _tc_mid(p0, W1, b1r, Ew1s, eb1s, e0)
    p1 = _edge_pass(h1, src3, dst3, w3)
    x2, energy = _tc_post(p1, Ew2s, eb2s, e1)
    return (energy, x2)


# speedup vs baseline: 4.9861x; 1.3292x over previous
import functools

import jax
import jax.numpy as jnp
from jax import lax
from jax.experimental import pallas as pl
from jax.experimental.pallas import tpu as pltpu
from jax.experimental.pallas import tpu_sc as plsc

N = 10000
D = 128
E = 320000

NC = 2
NS = 16
DH = D // NC
EPS = E // NS
CHUNK = 80
NCHUNK = EPS // CHUNK
NBUF = 5
N_PAD = 10240
RPT = N_PAD // NS

BN = 2000
GRID = N // BN


def _pre_body(x_ref, W_ref, b_ref, Ew_ref, eb_ref, h_ref, e_ref):
    xb = x_ref[...]
    h = jnp.dot(xb, W_ref[...], preferred_element_type=jnp.float32) + b_ref[...]
    h_ref[0] = h[:, :DH]
    h_ref[1] = h[:, DH:]
    e_ref[...] = (
        jnp.dot(xb, Ew_ref[...], preferred_element_type=jnp.float32) + eb_ref[...]
    )


_tc_pre = pl.pallas_call(
    _pre_body,
    grid=(GRID,),
    in_specs=[
        pl.BlockSpec((BN, D), lambda i: (i, 0)),
        pl.BlockSpec((D, D), lambda i: (0, 0)),
        pl.BlockSpec((1, D), lambda i: (0, 0)),
        pl.BlockSpec((D, 1), lambda i: (0, 0)),
        pl.BlockSpec((1, 1), lambda i: (0, 0)),
    ],
    out_specs=[
        pl.BlockSpec((2, BN, DH), lambda i: (0, i, 0)),
        pl.BlockSpec((BN, 1), lambda i: (i, 0)),
    ],
    out_shape=[
        jax.ShapeDtypeStruct((2, N, DH), jnp.float32),
        jax.ShapeDtypeStruct((N, 1), jnp.float32),
    ],
)


def _mid_body(p_ref, W_ref, b_ref, Ew_ref, eb_ref, e0_ref, h_ref, e_ref):
    s = jnp.concatenate([p_ref[0], p_ref[1]], axis=-1)
    x1 = jnp.where(s >= 0, s, 0.01 * s)
    h = jnp.dot(x1, W_ref[...], preferred_element_type=jnp.float32) + b_ref[...]
    h_ref[0] = h[:, :DH]
    h_ref[1] = h[:, DH:]
    e_ref[...] = (
        e0_ref[...]
        + jnp.dot(x1, Ew_ref[...], preferred_element_type=jnp.float32)
        + eb_ref[...]
    )


_tc_mid = pl.pallas_call(
    _mid_body,
    grid=(GRID,),
    in_specs=[
        pl.BlockSpec((2, BN, DH), lambda i: (0, i, 0)),
        pl.BlockSpec((D, D), lambda i: (0, 0)),
        pl.BlockSpec((1, D), lambda i: (0, 0)),
        pl.BlockSpec((D, 1), lambda i: (0, 0)),
        pl.BlockSpec((1, 1), lambda i: (0, 0)),
        pl.BlockSpec((BN, 1), lambda i: (i, 0)),
    ],
    out_specs=[
        pl.BlockSpec((2, BN, DH), lambda i: (0, i, 0)),
        pl.BlockSpec((BN, 1), lambda i: (i, 0)),
    ],
    out_shape=[
        jax.ShapeDtypeStruct((2, N, DH), jnp.float32),
        jax.ShapeDtypeStruct((N, 1), jnp.float32),
    ],
)


def _post_body(p_ref, Ew_ref, eb_ref, e1_ref, x_ref, e_ref):
    s = jnp.concatenate([p_ref[0], p_ref[1]], axis=-1)
    x2 = jnp.where(s >= 0, s, 0.01 * s)
    x_ref[...] = x2
    e_ref[...] = (
        e1_ref[...]
        + jnp.dot(x2, Ew_ref[...], preferred_element_type=jnp.float32)
        + eb_ref[...]
    )


_tc_post = pl.pallas_call(
    _post_body,
    grid=(GRID,),
    in_specs=[
        pl.BlockSpec((2, BN, DH), lambda i: (0, i, 0)),
        pl.BlockSpec((D, 1), lambda i: (0, 0)),
        pl.BlockSpec((1, 1), lambda i: (0, 0)),
        pl.BlockSpec((BN, 1), lambda i: (i, 0)),
    ],
    out_specs=[
        pl.BlockSpec((BN, D), lambda i: (i, 0)),
        pl.BlockSpec((BN, 1), lambda i: (i, 0)),
    ],
    out_shape=[
        jax.ShapeDtypeStruct((N, D), jnp.float32),
        jax.ShapeDtypeStruct((N, 1), jnp.float32),
    ],
)


_sc_mesh = plsc.VectorSubcoreMesh(core_axis_name="c", subcore_axis_name="s")


@functools.partial(
    pl.kernel,
    mesh=_sc_mesh,
    compiler_params=pltpu.CompilerParams(use_tc_tiling_on_sc=False),
    out_type=jax.ShapeDtypeStruct((NC, N_PAD, DH), jnp.float32),
    scratch_types=[
        pltpu.VMEM((NCHUNK, CHUNK), jnp.int32),
        pltpu.VMEM((NCHUNK, CHUNK), jnp.int32),
        pltpu.VMEM((NCHUNK, CHUNK), jnp.float32),
        pltpu.VMEM((NBUF * CHUNK, DH), jnp.float32),
        pltpu.VMEM_SHARED((N_PAD, DH), jnp.float32),
        pltpu.SemaphoreType.DMA,
        pltpu.SemaphoreType.DMA,
    ],
)
def _edge_pass(h_hbm, src_hbm, dst_hbm, w_hbm, out_hbm,
               srcv, dstv, wv, rows, acc, gsem, ssem):
    c = lax.axis_index("c")
    s = lax.axis_index("s")

    def zfill(i, carry):
        for l in range(DH // 16):
            rows[i, pl.ds(l * 16, 16)] = jnp.zeros((16,), jnp.float32)
        return carry

    lax.fori_loop(0, NBUF * CHUNK, zfill, 0)
    base = s * RPT
    pltpu.sync_copy(rows, acc.at[pl.ds(base, NBUF * CHUNK)])
    pltpu.sync_copy(rows.at[pl.ds(0, RPT - NBUF * CHUNK)],
                    acc.at[pl.ds(base + NBUF * CHUNK, RPT - NBUF * CHUNK)])
    plsc.subcore_barrier()

    pltpu.sync_copy(src_hbm.at[s], srcv)
    pltpu.sync_copy(dst_hbm.at[s], dstv)
    pltpu.sync_copy(w_hbm.at[s], wv)

    hc = h_hbm.at[c]

    def rslice(buf):
        return rows.at[pl.ds(buf * CHUNK, CHUNK)]

    def gather_start(j, buf):
        pltpu.async_copy(hc.at[srcv.at[j]], rslice(buf), gsem)

    def gather_wait(j, buf):
        pltpu.make_async_copy(hc.at[srcv.at[j]], rslice(buf), gsem).wait()

    def scatter_start(j, buf):
        pltpu.async_copy(rslice(buf), acc.at[dstv.at[j]], ssem, add=True)

    def scatter_wait(j, buf):
        pltpu.make_async_copy(rslice(buf), acc.at[dstv.at[j]], ssem).wait()

    def multiply(j, buf):
        def group(g, carry):
            w16 = wv[j, pl.ds(g * 16, 16)]
            for jj in range(16):
                e = g * 16 + jj
                w_e = jnp.full((16,), w16[jj], jnp.float32)
                r = buf * CHUNK + e
                for l in range(DH // 16):
                    sl = pl.ds(l * 16, 16)
                    rows[r, sl] = rows[r, sl] * w_e
            return carry

        lax.fori_loop(0, CHUNK // 16, group, 0)

    def step(j, k, ws, gs):
        gather_wait(j, k)
        multiply(j, k)
        scatter_start(j, k)
        if ws:
            scatter_wait(j - 2, (k + 3) % NBUF)
        if gs:
            gather_start(j + 3, (k + 3) % NBUF)

    gather_start(0, 0)
    gather_start(1, 1)
    gather_start(2, 2)
    for j in range(NBUF):
        step(j, j, j >= 2, True)

    def body(i, carry):
        j0 = NBUF * i
        for k in range(NBUF):
            step(j0 + k, k, True, True)
        return carry

    lax.fori_loop(1, NCHUNK // NBUF - 1, body, 0)

    j0 = NCHUNK - NBUF
    for k in range(NBUF):
        step(j0 + k, k, True, k < 2)
    scatter_wait(NCHUNK - 2, (NCHUNK - 2) % NBUF)
    scatter_wait(NCHUNK - 1, (NCHUNK - 1) % NBUF)

    plsc.subcore_barrier()
    pltpu.sync_copy(acc.at[pl.ds(base, RPT)], out_hbm.at[c, pl.ds(base, RPT)])


def kernel(x, edge_index, edge_w, temp, W0, b0, W1, b1, Ew0, eb0, Ew1, eb1, Ew2, eb2):
    src3 = edge_index[0].reshape(NS, NCHUNK, CHUNK)
    dst3 = edge_index[1].reshape(NS, NCHUNK, CHUNK)
    w3 = edge_w.reshape(NS, NCHUNK, CHUNK)

    b0r = b0.reshape(1, D)
    b1r = b1.reshape(1, D)
    Ew0s = Ew0 * temp[0]
    eb0s = (eb0 * temp[0]).reshape(1, 1)
    Ew1s = Ew1 * temp[1]
    eb1s = (eb1 * temp[1]).reshape(1, 1)
    Ew2s = Ew2 * temp[2]
    eb2s = (eb2 * temp[2]).reshape(1, 1)

    h0, e0 = _tc_pre(x, W0, b0r, Ew0s, eb0s)
    p0 = _edge_pass(h0, src3, dst3, w3)
    h1, e1 = _tc_mid(p0, W1, b1r, Ew1s, eb1s, e0)
    p1 = _edge_pass(h1, src3, dst3, w3)
    x2, energy = _tc_post(p1, Ew2s, eb2s, e1)
    return (energy, x2)


# speedup vs baseline: 10.1029x; 2.0262x over previous
import functools

import jax
import jax.numpy as jnp
from jax import lax
from jax.experimental import pallas as pl
from jax.experimental.pallas import tpu as pltpu
from jax.experimental.pallas import tpu_sc as plsc

N = 10000
D = 128
E = 320000

NC = 2
NS = 16
DH = D // NC
EPS = E // NS
CHUNK = 80
NCHUNK = EPS // CHUNK
NBUF = 5
N_PAD = 10240
RPT = N_PAD // NS

BN = 2000
GRID = N // BN


def _pre_body(x_ref, W_ref, b_ref, Ew_ref, eb_ref, h_ref, e_ref):
    xb = x_ref[...]
    h = jnp.dot(xb, W_ref[...], preferred_element_type=jnp.float32) + b_ref[...]
    h_ref[0] = h[:, :DH]
    h_ref[1] = h[:, DH:]
    e_ref[...] = (
        jnp.dot(xb, Ew_ref[...], preferred_element_type=jnp.float32) + eb_ref[...]
    )


_tc_pre = pl.pallas_call(
    _pre_body,
    grid=(GRID,),
    in_specs=[
        pl.BlockSpec((BN, D), lambda i: (i, 0)),
        pl.BlockSpec((D, D), lambda i: (0, 0)),
        pl.BlockSpec((1, D), lambda i: (0, 0)),
        pl.BlockSpec((D, 1), lambda i: (0, 0)),
        pl.BlockSpec((1, 1), lambda i: (0, 0)),
    ],
    out_specs=[
        pl.BlockSpec((2, BN, DH), lambda i: (0, i, 0)),
        pl.BlockSpec((BN, 1), lambda i: (i, 0)),
    ],
    out_shape=[
        jax.ShapeDtypeStruct((2, N, DH), jnp.float32),
        jax.ShapeDtypeStruct((N, 1), jnp.float32),
    ],
)


def _mid_body(p_ref, W_ref, b_ref, Ew_ref, eb_ref, e0_ref, h_ref, e_ref):
    s = jnp.concatenate([p_ref[0], p_ref[1]], axis=-1)
    x1 = jnp.where(s >= 0, s, 0.01 * s)
    h = jnp.dot(x1, W_ref[...], preferred_element_type=jnp.float32) + b_ref[...]
    h_ref[0] = h[:, :DH]
    h_ref[1] = h[:, DH:]
    e_ref[...] = (
        e0_ref[...]
        + jnp.dot(x1, Ew_ref[...], preferred_element_type=jnp.float32)
        + eb_ref[...]
    )


_tc_mid = pl.pallas_call(
    _mid_body,
    grid=(GRID,),
    in_specs=[
        pl.BlockSpec((2, BN, DH), lambda i: (0, i, 0)),
        pl.BlockSpec((D, D), lambda i: (0, 0)),
        pl.BlockSpec((1, D), lambda i: (0, 0)),
        pl.BlockSpec((D, 1), lambda i: (0, 0)),
        pl.BlockSpec((1, 1), lambda i: (0, 0)),
        pl.BlockSpec((BN, 1), lambda i: (i, 0)),
    ],
    out_specs=[
        pl.BlockSpec((2, BN, DH), lambda i: (0, i, 0)),
        pl.BlockSpec((BN, 1), lambda i: (i, 0)),
    ],
    out_shape=[
        jax.ShapeDtypeStruct((2, N, DH), jnp.float32),
        jax.ShapeDtypeStruct((N, 1), jnp.float32),
    ],
)


def _post_body(p_ref, Ew_ref, eb_ref, e1_ref, x_ref, e_ref):
    s = jnp.concatenate([p_ref[0], p_ref[1]], axis=-1)
    x2 = jnp.where(s >= 0, s, 0.01 * s)
    x_ref[...] = x2
    e_ref[...] = (
        e1_ref[...]
        + jnp.dot(x2, Ew_ref[...], preferred_element_type=jnp.float32)
        + eb_ref[...]
    )


_tc_post = pl.pallas_call(
    _post_body,
    grid=(GRID,),
    in_specs=[
        pl.BlockSpec((2, BN, DH), lambda i: (0, i, 0)),
        pl.BlockSpec((D, 1), lambda i: (0, 0)),
        pl.BlockSpec((1, 1), lambda i: (0, 0)),
        pl.BlockSpec((BN, 1), lambda i: (i, 0)),
    ],
    out_specs=[
        pl.BlockSpec((BN, D), lambda i: (i, 0)),
        pl.BlockSpec((BN, 1), lambda i: (i, 0)),
    ],
    out_shape=[
        jax.ShapeDtypeStruct((N, D), jnp.float32),
        jax.ShapeDtypeStruct((N, 1), jnp.float32),
    ],
)


_sc_mesh = plsc.VectorSubcoreMesh(core_axis_name="c", subcore_axis_name="s")


@functools.partial(
    pl.kernel,
    mesh=_sc_mesh,
    compiler_params=pltpu.CompilerParams(use_tc_tiling_on_sc=False),
    out_type=jax.ShapeDtypeStruct((NC, N_PAD, DH), jnp.float32),
    scratch_types=[
        pltpu.VMEM((NCHUNK, CHUNK), jnp.int32),
        pltpu.VMEM((NCHUNK, CHUNK), jnp.int32),
        pltpu.VMEM((NCHUNK, CHUNK), jnp.float32),
        pltpu.VMEM((NBUF * CHUNK, DH), jnp.float32),
        pltpu.VMEM_SHARED((N_PAD, DH), jnp.float32),
        pltpu.SemaphoreType.DMA,
        pltpu.SemaphoreType.DMA,
    ],
)
def _edge_pass(h_hbm, src_hbm, dst_hbm, w_hbm, out_hbm,
               srcv, dstv, wv, rows, acc, gsem, ssem):
    c = lax.axis_index("c")
    s = lax.axis_index("s")

    def zfill(i, carry):
        for l in range(DH // 16):
            rows[i, pl.ds(l * 16, 16)] = jnp.zeros((16,), jnp.float32)
        return carry

    lax.fori_loop(0, NBUF * CHUNK, zfill, 0)
    base = s * RPT
    pltpu.sync_copy(rows, acc.at[pl.ds(base, NBUF * CHUNK)])
    pltpu.sync_copy(rows.at[pl.ds(0, RPT - NBUF * CHUNK)],
                    acc.at[pl.ds(base + NBUF * CHUNK, RPT - NBUF * CHUNK)])
    plsc.subcore_barrier()

    pltpu.sync_copy(src_hbm.at[s], srcv)
    pltpu.sync_copy(dst_hbm.at[s], dstv)
    pltpu.sync_copy(w_hbm.at[s], wv)

    hc = h_hbm.at[c]

    def rslice(buf):
        return rows.at[pl.ds(buf * CHUNK, CHUNK)]

    def gather_start(j, buf):
        pltpu.async_copy(hc.at[srcv.at[j]], rslice(buf), gsem)

    def gather_wait(j, buf):
        pltpu.make_async_copy(hc.at[srcv.at[j]], rslice(buf), gsem).wait()

    def scatter_start(j, buf):
        pltpu.async_copy(rslice(buf), acc.at[dstv.at[j]], ssem, add=True)

    def scatter_wait(j, buf):
        pltpu.make_async_copy(rslice(buf), acc.at[dstv.at[j]], ssem).wait()

    def multiply(j, buf):
        rbase = buf * CHUNK
        for g in range(CHUNK // 16):
            w16 = wv[j, pl.ds(g * 16, 16)]
            for jj in range(16):
                w_e = jnp.take_along_axis(
                    w16, jnp.full((16,), jj, jnp.int32), axis=0,
                    mode=lax.GatherScatterMode.PROMISE_IN_BOUNDS)
                r = rbase + g * 16 + jj
                for l in range(DH // 16):
                    sl = pl.ds(l * 16, 16)
                    rows[r, sl] = rows[r, sl] * w_e

    gather_start(0, 0)
    gather_start(1, 1)
    gather_start(2, 2)

    def body(j, carry):
        buf = lax.rem(j, NBUF)
        gather_wait(j, buf)
        multiply(j, buf)
        scatter_start(j, buf)

        @pl.when(j >= 2)
        def _():
            scatter_wait(j - 2, lax.rem(j + 3, NBUF))

        @pl.when(j < NCHUNK - 3)
        def _():
            gather_start(j + 3, lax.rem(j + 3, NBUF))

        return carry

    lax.fori_loop(0, NCHUNK, body, 0)
    scatter_wait(NCHUNK - 2, (NCHUNK - 2) % NBUF)
    scatter_wait(NCHUNK - 1, (NCHUNK - 1) % NBUF)

    plsc.subcore_barrier()
    pltpu.sync_copy(acc.at[pl.ds(base, RPT)], out_hbm.at[c, pl.ds(base, RPT)])


def kernel(x, edge_index, edge_w, temp, W0, b0, W1, b1, Ew0, eb0, Ew1, eb1, Ew2, eb2):
    src3 = edge_index[0].reshape(NS, NCHUNK, CHUNK)
    dst3 = edge_index[1].reshape(NS, NCHUNK, CHUNK)
    w3 = edge_w.reshape(NS, NCHUNK, CHUNK)

    b0r = b0.reshape(1, D)
    b1r = b1.reshape(1, D)
    Ew0s = Ew0 * temp[0]
    eb0s = (eb0 * temp[0]).reshape(1, 1)
    Ew1s = Ew1 * temp[1]
    eb1s = (eb1 * temp[1]).reshape(1, 1)
    Ew2s = Ew2 * temp[2]
    eb2s = (eb2 * temp[2]).reshape(1, 1)

    h0, e0 = _tc_pre(x, W0, b0r, Ew0s, eb0s)
    p0 = _edge_pass(h0, src3, dst3, w3)
    h1, e1 = _tc_mid(p0, W1, b1r, Ew1s, eb1s, e0)
    p1 = _edge_pass(h1, src3, dst3, w3)
    x2, energy = _tc_post(p1, Ew2s, eb2s, e1)
    return (energy, x2)
